# baseline (speedup 1.0000x reference)
import functools
import numpy as np
import jax
import jax.numpy as jnp
from jax import lax
from jax.experimental import pallas as pl
from jax.experimental.pallas import tpu as pltpu
from jax.experimental.pallas import tpu_sc as plsc

N = 10000
E = 320000
FD = 128
NUM_REL = 3
IDX = 2000

NC = 2
NS = 16
NW = NC * NS

CH = 128
EPW = 10240
E_PAD = EPW * NW
N_CHUNKS = EPW // CH

CNT_PAD = 30720
ACC_ROWS = 10240
IDX_PAD = 2048
IPW = IDX_PAD // NW

_Z = np.int32(0)

_mesh = plsc.VectorSubcoreMesh(core_axis_name="c", subcore_axis_name="s")


def _wid():
  return lax.axis_index("c") * NS + lax.axis_index("s")


def _loop32(n, unroll=None):
  del unroll
  return pl.loop(jnp.int32(0), jnp.int32(n))


def _zero_fill(buf, n16):
  z = jnp.zeros((16,), jnp.float32)
  if buf.ndim == 1:
    @_loop32(n16)
    def _(i):
      buf[pl.ds(i * 16, 16)] = z
  else:
    rows, cols = buf.shape

    @_loop32(rows)
    def _(i):
      for f in range(cols // 16):
        buf[i, pl.ds(f * 16, 16)] = z


@functools.partial(
    pl.kernel,
    out_type=jax.ShapeDtypeStruct((NC, CNT_PAD), jnp.float32),
    mesh=_mesh,
    compiler_params=pltpu.CompilerParams(needs_layout_passes=False),
    scratch_types=[
        pltpu.MemorySpace.VMEM_SHARED((CNT_PAD,), jnp.float32),
        pltpu.VMEM((2, CH), jnp.int32),
        pltpu.VMEM((CH,), jnp.float32),
        pltpu.VMEM((1920,), jnp.float32),
        [pltpu.SemaphoreType.DMA] * 2,
        pltpu.SemaphoreType.DMA,
    ],
)
def _count_kernel(cidx_hbm, cnt2_hbm, cnt_sh, idxb, ones_v, zbuf, lsem, ssem):
  c = lax.axis_index("c")
  s = lax.axis_index("s")
  _zero_fill(zbuf, 120)
  pltpu.sync_copy(zbuf, cnt_sh.at[pl.ds(s * 1920, 1920)])
  one = jnp.ones((16,), jnp.float32)
  for g in range(CH // 16):
    ones_v[pl.ds(g * 16, 16)] = one
  plsc.subcore_barrier()

  w = _wid()
  base_w = w * EPW

  def bload(slot, j):
    pltpu.async_copy(cidx_hbm.at[pl.ds(base_w + j * CH, CH)],
                     idxb.at[jnp.int32(slot)], lsem[slot])

  def bload_wait(slot):
    pltpu.make_async_copy(cidx_hbm.at[pl.ds(base_w, CH)],
                          idxb.at[jnp.int32(slot)], lsem[slot]).wait()

  def bscat_wait(slot):
    pltpu.make_async_copy(ones_v, cnt_sh.at[idxb.at[jnp.int32(slot)]],
                          ssem).wait()

  bload(0, jnp.int32(0))

  @_loop32(N_CHUNKS // 2)
  def _(j2):
    for b in (0, 1):
      j = j2 * 2 + b
      bload_wait(b)

      @pl.when(j >= 1)
      def _():
        bscat_wait(1 - b)

      @pl.when(j + 1 < N_CHUNKS)
      def _():
        bload(1 - b, j + 1)

      pltpu.async_copy(ones_v, cnt_sh.at[idxb.at[jnp.int32(b)]], ssem,
                       add=True)

  bscat_wait(1)
  plsc.subcore_barrier()
  pltpu.sync_copy(cnt_sh.at[pl.ds(s * 1920, 1920)],
                  cnt2_hbm.at[c, pl.ds(s * 1920, 1920)])


def _rc_body(cnt_ref, rc_ref):
  t = cnt_ref[0] + cnt_ref[1]
  rc_ref[...] = 1.0 / jnp.maximum(t, 1.0)


def _rc_table(cnt2):
  return pl.pallas_call(
      _rc_body,
      grid=(1,),
      in_specs=[pl.BlockSpec((NC, CNT_PAD // 128, 128),
                             lambda i: (_Z, _Z, _Z))],
      out_specs=pl.BlockSpec((CNT_PAD // 128, 128), lambda i: (_Z, _Z)),
      out_shape=jax.ShapeDtypeStruct((CNT_PAD // 128, 128), jnp.float32),
  )(cnt2)


@functools.partial(
    pl.kernel,
    out_type=jax.ShapeDtypeStruct((NC, ACC_ROWS, FD), jnp.float32),
    mesh=_mesh,
    compiler_params=pltpu.CompilerParams(needs_layout_passes=False),
    scratch_types=[
        pltpu.MemorySpace.VMEM_SHARED((ACC_ROWS, FD), jnp.float32),
        pltpu.VMEM((4, CH), jnp.int32),
        pltpu.VMEM((4, CH), jnp.int32),
        pltpu.VMEM((4, CH), jnp.int32),
        pltpu.VMEM((4, CH), jnp.float32),
        pltpu.VMEM((2, CH, FD), jnp.float32),
        pltpu.VMEM((16, FD), jnp.float32),
        [pltpu.SemaphoreType.DMA] * 4,
        [pltpu.SemaphoreType.DMA] * 4,
        [pltpu.SemaphoreType.DMA] * 2,
        pltpu.SemaphoreType.DMA,
    ],
)
def _edge_kernel(y_hbm, gidx_hbm, didx_hbm, cidx_hbm, rc_hbm, acc_hbm,
                 acc_sh, gi_v, di_v, ci_v, w_v, rows_v, zbuf,
                 isem, wsem, gsem, ssem):
  c = lax.axis_index("c")
  s = lax.axis_index("s")
  _zero_fill(zbuf, 128)

  @_loop32(40)
  def _(k):
    pltpu.sync_copy(zbuf, acc_sh.at[pl.ds(s * 640 + k * 16, 16), :])

  plsc.subcore_barrier()

  w = _wid()
  base_w = w * EPW

  def idx_load(s4, j):
    si = jnp.int32(s4)
    b = base_w + j * CH
    pltpu.async_copy(gidx_hbm.at[pl.ds(b, CH)], gi_v.at[si], isem[s4])
    pltpu.async_copy(didx_hbm.at[pl.ds(b, CH)], di_v.at[si], isem[s4])
    pltpu.async_copy(cidx_hbm.at[pl.ds(b, CH)], ci_v.at[si], isem[s4])

  def idx_wait(s4):
    si = jnp.int32(s4)
    b0 = pl.ds(base_w, CH)
    pltpu.make_async_copy(gidx_hbm.at[b0], gi_v.at[si], isem[s4]).wait()
    pltpu.make_async_copy(didx_hbm.at[b0], di_v.at[si], isem[s4]).wait()
    pltpu.make_async_copy(cidx_hbm.at[b0], ci_v.at[si], isem[s4]).wait()

  def wg_start(s4):
    si = jnp.int32(s4)
    pltpu.async_copy(rc_hbm.at[ci_v.at[si]], w_v.at[si], wsem[s4])

  def wg_wait(s4):
    si = jnp.int32(s4)
    pltpu.make_async_copy(rc_hbm.at[ci_v.at[si]], w_v.at[si],
                          wsem[s4]).wait()

  def gather_start(s4, s2):
    pltpu.async_copy(y_hbm.at[gi_v.at[jnp.int32(s4)]],
                     rows_v.at[jnp.int32(s2)], gsem[s2])

  def gather_wait(s4, s2):
    pltpu.make_async_copy(y_hbm.at[gi_v.at[jnp.int32(s4)]],
                          rows_v.at[jnp.int32(s2)], gsem[s2]).wait()

  def scatter_wait(s4, s2):
    pltpu.make_async_copy(rows_v.at[jnp.int32(s2)],
                          acc_sh.at[di_v.at[jnp.int32(s4)]], ssem).wait()

  idx_load(0, jnp.int32(0))
  idx_load(1, jnp.int32(1))
  idx_wait(0)
  wg_start(0)
  gather_start(0, 0)

  @_loop32(N_CHUNKS // 4)
  def _(j4):
    for b in range(4):
      j = j4 * 4 + b
      s2 = b % 2
      gather_wait(b, s2)

      @pl.when(j + 2 < N_CHUNKS)
      def _():
        idx_load((b + 2) % 4, j + 2)

      @pl.when(j + 1 < N_CHUNKS)
      def _():
        idx_wait((b + 1) % 4)
        wg_start((b + 1) % 4)

      @pl.when(j >= 1)
      def _():
        scatter_wait((b - 1) % 4, 1 - s2)

      @pl.when(j + 1 < N_CHUNKS)
      def _():
        gather_start((b + 1) % 4, 1 - s2)

      wg_wait(b)
      bi = jnp.int32(s2)
      rv = rows_v.at[bi]
      wv = w_v.at[jnp.int32(b)]

      @_loop32(CH // 8)
      def _(r8):
        for u in range(8):
          r = r8 * 8 + u
          ws = plsc.load_gather(wv, [jnp.zeros((16,), jnp.int32) + r])
          for f in range(FD // 16):
            rv[r, pl.ds(f * 16, 16)] = rv[r, pl.ds(f * 16, 16)] * ws

      pltpu.async_copy(rows_v.at[bi], acc_sh.at[di_v.at[jnp.int32(b)]], ssem,
                       add=True)

  scatter_wait(3, 1)
  plsc.subcore_barrier()
  pltpu.sync_copy(acc_sh.at[pl.ds(s * 640, 640), :],
                  acc_hbm.at[c, pl.ds(s * 640, 640), :])


@functools.partial(
    pl.kernel,
    out_type=jax.ShapeDtypeStruct((IDX_PAD, FD), jnp.float32),
    mesh=_mesh,
    compiler_params=pltpu.CompilerParams(needs_layout_passes=False),
    scratch_types=[
        pltpu.VMEM((IPW,), jnp.int32),
        pltpu.VMEM((IPW, FD), jnp.float32),
        pltpu.VMEM((IPW, FD), jnp.float32),
        pltpu.VMEM((IPW, FD), jnp.float32),
        pltpu.SemaphoreType.DMA,
        pltpu.SemaphoreType.DMA,
        pltpu.SemaphoreType.DMA,
    ],
)
def _final_gather_kernel(r2_hbm, acc_hbm, idx_hbm, out_hbm,
                         idx_v, a_v, b_v, c_v, sem0, sem1, sem2):
  w = _wid()
  base = w * IPW
  pltpu.sync_copy(idx_hbm.at[pl.ds(base, IPW)], idx_v)
  cp0 = pltpu.async_copy(r2_hbm.at[idx_v], a_v, sem0)
  cp1 = pltpu.async_copy(acc_hbm.at[jnp.int32(0)].at[idx_v], b_v, sem1)
  cp2 = pltpu.async_copy(acc_hbm.at[jnp.int32(1)].at[idx_v], c_v, sem2)
  cp0.wait()
  cp1.wait()
  cp2.wait()

  @_loop32(IPW)
  def _(r):
    for f in range(FD // 16):
      sl = pl.ds(f * 16, 16)
      a_v[r, sl] = a_v[r, sl] + b_v[r, sl] + c_v[r, sl]

  pltpu.sync_copy(a_v, out_hbm.at[pl.ds(base, IPW)])


_BLK = 400
_GRID = N // _BLK


def _enc_body(vf_ref, tf_ref, w1_ref, b1_ref, w2_ref, b2_ref, wc_ref, bc_ref,
              out_ref):
  hp = jax.lax.Precision.HIGHEST
  v = jnp.dot(vf_ref[...], w1_ref[...], precision=hp) + b1_ref[...]
  t = jnp.dot(tf_ref[...], w2_ref[...], precision=hp) + b2_ref[...]
  z = (jnp.dot(v, wc_ref[0:FD, :], precision=hp)
       + jnp.dot(t, wc_ref[FD:2 * FD, :], precision=hp) + bc_ref[...])
  out_ref[...] = jnp.where(z >= 0, z, 0.01 * z)


def _encoder(vf, tf, W1, b1, W2, b2, Wc, bc):
  return pl.pallas_call(
      _enc_body,
      grid=(_GRID,),
      in_specs=[
          pl.BlockSpec((_BLK, 8), lambda i: (i, _Z)),
          pl.BlockSpec((_BLK, 768), lambda i: (i, _Z)),
          pl.BlockSpec((8, FD), lambda i: (_Z, _Z)),
          pl.BlockSpec((1, FD), lambda i: (_Z, _Z)),
          pl.BlockSpec((768, FD), lambda i: (_Z, _Z)),
          pl.BlockSpec((1, FD), lambda i: (_Z, _Z)),
          pl.BlockSpec((2 * FD, FD), lambda i: (_Z, _Z)),
          pl.BlockSpec((1, FD), lambda i: (_Z, _Z)),
      ],
      out_specs=pl.BlockSpec((_BLK, FD), lambda i: (i, _Z)),
      out_shape=jax.ShapeDtypeStruct((N, FD), jnp.float32),
  )(vf, tf, W1, b1, W2, b2, Wc, bc)


def _layer_body(n_acc, x_ref, acc_ref, wrel_ref, wroot_ref, b_ref,
                y_ref, r_ref):
  hp = jax.lax.Precision.HIGHEST
  x = x_ref[...]
  if n_acc:
    x = x + acc_ref[0] + acc_ref[1]
  for r in range(NUM_REL):
    y_ref[r] = jnp.dot(x, wrel_ref[r], precision=hp)
  r_ref[...] = jnp.dot(x, wroot_ref[...], precision=hp) + b_ref[...]


def _layer_dense(x, acc, Wrel, Wroot, b):
  n_acc = acc is not None
  in_specs = [pl.BlockSpec((_BLK, FD), lambda i: (i, _Z))]
  args = [x]
  if n_acc:
    in_specs.append(pl.BlockSpec((NC, _BLK, FD), lambda i: (_Z, i, _Z)))
    args.append(acc)
  else:
    in_specs.append(pl.BlockSpec((1, 1), lambda i: (_Z, _Z)))
    args.append(jnp.zeros((1, 1), jnp.float32))
  in_specs += [
      pl.BlockSpec((NUM_REL, FD, FD), lambda i: (_Z, _Z, _Z)),
      pl.BlockSpec((FD, FD), lambda i: (_Z, _Z)),
      pl.BlockSpec((1, FD), lambda i: (_Z, _Z)),
  ]
  return pl.pallas_call(
      functools.partial(_layer_body, n_acc),
      grid=(_GRID,),
      in_specs=in_specs,
      out_specs=[
          pl.BlockSpec((NUM_REL, _BLK, FD), lambda i: (_Z, i, _Z)),
          pl.BlockSpec((_BLK, FD), lambda i: (i, _Z)),
      ],
      out_shape=[
          jax.ShapeDtypeStruct((NUM_REL, N, FD), jnp.float32),
          jax.ShapeDtypeStruct((N, FD), jnp.float32),
      ],
  )(*args, Wrel, Wroot, b)


def _head_body(g_ref, w3_ref, b3_ref, out_ref):
  out_ref[...] = (jnp.dot(g_ref[...], w3_ref[...],
                          precision=jax.lax.Precision.HIGHEST) + b3_ref[...])


def _head(g, W3, b3):
  cl = W3.shape[1]
  return pl.pallas_call(
      _head_body,
      grid=(1,),
      in_specs=[
          pl.BlockSpec((IDX_PAD, FD), lambda i: (_Z, _Z)),
          pl.BlockSpec((FD, cl), lambda i: (_Z, _Z)),
          pl.BlockSpec((1, cl), lambda i: (_Z, _Z)),
      ],
      out_specs=pl.BlockSpec((IDX_PAD, cl), lambda i: (_Z, _Z)),
      out_shape=jax.ShapeDtypeStruct((IDX_PAD, cl), jnp.float32),
  )(g, W3, b3)


def kernel(value_feature, text_feature, edge_index, edge_type, idx,
           W1, b1, W2, b2, Wc, bc,
           Wrel1, Wroot1, brg1, Wrel2, Wroot2, brg2, W3, b3):
  src = edge_index[0].astype(jnp.int32)
  dst = edge_index[1].astype(jnp.int32)
  et = edge_type.astype(jnp.int32)

  pad = E_PAD - E
  gidx = jnp.concatenate([et * N + src, jnp.zeros((pad,), jnp.int32)])
  didx = jnp.concatenate([dst, jnp.full((pad,), N, jnp.int32)])
  cidx = jnp.concatenate([et * N + dst, jnp.full((pad,), 3 * N, jnp.int32)])
  idxp = jnp.concatenate(
      [idx.astype(jnp.int32), jnp.zeros((IDX_PAD - IDX,), jnp.int32)])

  b1r = b1.reshape(1, FD)
  b2r = b2.reshape(1, FD)
  bcr = bc.reshape(1, FD)
  brg1r = brg1.reshape(1, FD)
  brg2r = brg2.reshape(1, FD)
  b3r = b3.reshape(1, -1)

  cnt2 = _count_kernel(cidx)
  rc = _rc_table(cnt2.reshape(NC, CNT_PAD // 128, 128)).reshape(CNT_PAD)

  f0 = _encoder(value_feature, text_feature, W1, b1r, W2, b2r, Wc, bcr)

  y1, r1 = _layer_dense(f0, None, Wrel1, Wroot1, brg1r)
  acc1 = _edge_kernel(y1.reshape(NUM_REL * N, FD), gidx, didx, cidx, rc)

  y2, r2 = _layer_dense(r1, acc1, Wrel2, Wroot2, brg2r)
  acc2 = _edge_kernel(y2.reshape(NUM_REL * N, FD), gidx, didx, cidx, rc)

  g = _final_gather_kernel(r2, acc2, idxp)
  out = _head(g, W3, b3r)
  return out[:IDX]


# speedup vs baseline: 2.5547x; 2.5547x over previous
import functools
import numpy as np
import jax
import jax.numpy as jnp
from jax import lax
from jax.experimental import pallas as pl
from jax.experimental.pallas import tpu as pltpu
from jax.experimental.pallas import tpu_sc as plsc

N = 10000
E = 320000
FD = 128
NUM_REL = 3
IDX = 2000

NC = 2
NS = 16
NW = NC * NS

CH = 128
EPW = 10240
E_PAD = EPW * NW
N_CHUNKS = EPW // CH

CNT_PAD = 30720
ACC_ROWS = 10240
IDX_PAD = 2048
IPW = IDX_PAD // NW

_Z = np.int32(0)

_mesh = plsc.VectorSubcoreMesh(core_axis_name="c", subcore_axis_name="s")


def _wid():
  return lax.axis_index("c") * NS + lax.axis_index("s")


def _loop32(n, unroll=None):
  del unroll
  return pl.loop(jnp.int32(0), jnp.int32(n))


def _zero_fill(buf, n16):
  z = jnp.zeros((16,), jnp.float32)
  if buf.ndim == 1:
    @_loop32(n16)
    def _(i):
      buf[pl.ds(i * 16, 16)] = z
  else:
    rows, cols = buf.shape

    @_loop32(rows)
    def _(i):
      for f in range(cols // 16):
        buf[i, pl.ds(f * 16, 16)] = z


@functools.partial(
    pl.kernel,
    out_type=jax.ShapeDtypeStruct((NC, CNT_PAD), jnp.float32),
    mesh=_mesh,
    compiler_params=pltpu.CompilerParams(needs_layout_passes=False),
    scratch_types=[
        pltpu.MemorySpace.VMEM_SHARED((CNT_PAD,), jnp.float32),
        pltpu.VMEM((2, CH), jnp.int32),
        pltpu.VMEM((CH,), jnp.float32),
        pltpu.VMEM((1920,), jnp.float32),
        [pltpu.SemaphoreType.DMA] * 2,
        pltpu.SemaphoreType.DMA,
    ],
)
def _count_kernel(cidx_hbm, cnt2_hbm, cnt_sh, idxb, ones_v, zbuf, lsem, ssem):
  c = lax.axis_index("c")
  s = lax.axis_index("s")
  _zero_fill(zbuf, 120)
  pltpu.sync_copy(zbuf, cnt_sh.at[pl.ds(s * 1920, 1920)])
  one = jnp.ones((16,), jnp.float32)
  for g in range(CH // 16):
    ones_v[pl.ds(g * 16, 16)] = one
  plsc.subcore_barrier()

  w = _wid()
  base_w = w * EPW

  def bload(slot, j):
    pltpu.async_copy(cidx_hbm.at[pl.ds(base_w + j * CH, CH)],
                     idxb.at[jnp.int32(slot)], lsem[slot])

  def bload_wait(slot):
    pltpu.make_async_copy(cidx_hbm.at[pl.ds(base_w, CH)],
                          idxb.at[jnp.int32(slot)], lsem[slot]).wait()

  def bscat_wait(slot):
    pltpu.make_async_copy(ones_v, cnt_sh.at[idxb.at[jnp.int32(slot)]],
                          ssem).wait()

  bload(0, jnp.int32(0))

  @_loop32(N_CHUNKS // 2)
  def _(j2):
    for b in (0, 1):
      j = j2 * 2 + b
      bload_wait(b)

      @pl.when(j >= 1)
      def _():
        bscat_wait(1 - b)

      @pl.when(j + 1 < N_CHUNKS)
      def _():
        bload(1 - b, j + 1)

      pltpu.async_copy(ones_v, cnt_sh.at[idxb.at[jnp.int32(b)]], ssem,
                       add=True)

  bscat_wait(1)
  plsc.subcore_barrier()
  pltpu.sync_copy(cnt_sh.at[pl.ds(s * 1920, 1920)],
                  cnt2_hbm.at[c, pl.ds(s * 1920, 1920)])


def _rc_body(cnt_ref, rc_ref):
  t = cnt_ref[0] + cnt_ref[1]
  rc_ref[...] = 1.0 / jnp.maximum(t, 1.0)


def _rc_table(cnt2):
  return pl.pallas_call(
      _rc_body,
      grid=(1,),
      in_specs=[pl.BlockSpec((NC, CNT_PAD // 128, 128),
                             lambda i: (_Z, _Z, _Z))],
      out_specs=pl.BlockSpec((CNT_PAD // 128, 128), lambda i: (_Z, _Z)),
      out_shape=jax.ShapeDtypeStruct((CNT_PAD // 128, 128), jnp.float32),
  )(cnt2)


@functools.partial(
    pl.kernel,
    out_type=jax.ShapeDtypeStruct((NC, ACC_ROWS, FD), jnp.float32),
    mesh=_mesh,
    compiler_params=pltpu.CompilerParams(needs_layout_passes=False),
    scratch_types=[
        pltpu.MemorySpace.VMEM_SHARED((ACC_ROWS, FD), jnp.float32),
        pltpu.VMEM((4, CH), jnp.int32),
        pltpu.VMEM((4, CH), jnp.int32),
        pltpu.VMEM((4, CH), jnp.int32),
        pltpu.VMEM((4, CH), jnp.float32),
        pltpu.VMEM((2, CH, FD), jnp.float32),
        pltpu.VMEM((16, FD), jnp.float32),
        [pltpu.SemaphoreType.DMA] * 4,
        [pltpu.SemaphoreType.DMA] * 4,
        [pltpu.SemaphoreType.DMA] * 2,
        pltpu.SemaphoreType.DMA,
    ],
)
def _edge_kernel(y_hbm, gidx_hbm, didx_hbm, cidx_hbm, rc_hbm, acc_hbm,
                 acc_sh, gi_v, di_v, ci_v, w_v, rows_v, zbuf,
                 isem, wsem, gsem, ssem):
  c = lax.axis_index("c")
  s = lax.axis_index("s")
  _zero_fill(zbuf, 128)

  @_loop32(40)
  def _(k):
    pltpu.sync_copy(zbuf, acc_sh.at[pl.ds(s * 640 + k * 16, 16), :])

  plsc.subcore_barrier()

  w = _wid()
  base_w = w * EPW

  def idx_load(s4, j):
    si = jnp.int32(s4)
    b = base_w + j * CH
    pltpu.async_copy(gidx_hbm.at[pl.ds(b, CH)], gi_v.at[si], isem[s4])
    pltpu.async_copy(didx_hbm.at[pl.ds(b, CH)], di_v.at[si], isem[s4])
    pltpu.async_copy(cidx_hbm.at[pl.ds(b, CH)], ci_v.at[si], isem[s4])

  def idx_wait(s4):
    si = jnp.int32(s4)
    b0 = pl.ds(base_w, CH)
    pltpu.make_async_copy(gidx_hbm.at[b0], gi_v.at[si], isem[s4]).wait()
    pltpu.make_async_copy(didx_hbm.at[b0], di_v.at[si], isem[s4]).wait()
    pltpu.make_async_copy(cidx_hbm.at[b0], ci_v.at[si], isem[s4]).wait()

  def wg_start(s4):
    si = jnp.int32(s4)
    pltpu.async_copy(rc_hbm.at[ci_v.at[si]], w_v.at[si], wsem[s4])

  def wg_wait(s4):
    si = jnp.int32(s4)
    pltpu.make_async_copy(rc_hbm.at[ci_v.at[si]], w_v.at[si],
                          wsem[s4]).wait()

  def gather_start(s4, s2):
    pltpu.async_copy(y_hbm.at[gi_v.at[jnp.int32(s4)]],
                     rows_v.at[jnp.int32(s2)], gsem[s2])

  def gather_wait(s4, s2):
    pltpu.make_async_copy(y_hbm.at[gi_v.at[jnp.int32(s4)]],
                          rows_v.at[jnp.int32(s2)], gsem[s2]).wait()

  def scatter_wait(s4, s2):
    pltpu.make_async_copy(rows_v.at[jnp.int32(s2)],
                          acc_sh.at[di_v.at[jnp.int32(s4)]], ssem).wait()

  idx_load(0, jnp.int32(0))
  idx_load(1, jnp.int32(1))
  idx_wait(0)
  wg_start(0)
  gather_start(0, 0)

  @_loop32(N_CHUNKS // 4)
  def _(j4):
    for b in range(4):
      j = j4 * 4 + b
      s2 = b % 2
      gather_wait(b, s2)

      @pl.when(j + 2 < N_CHUNKS)
      def _():
        idx_load((b + 2) % 4, j + 2)

      @pl.when(j + 1 < N_CHUNKS)
      def _():
        idx_wait((b + 1) % 4)
        wg_start((b + 1) % 4)

      @pl.when(j >= 1)
      def _():
        scatter_wait((b - 1) % 4, 1 - s2)

      @pl.when(j + 1 < N_CHUNKS)
      def _():
        gather_start((b + 1) % 4, 1 - s2)

      wg_wait(b)
      bi = jnp.int32(s2)
      rv = rows_v.at[bi]
      wv = w_v.at[jnp.int32(b)]

      @_loop32(CH // 8)
      def _(r8):
        for u in range(8):
          r = r8 * 8 + u
          ws = plsc.load_gather(wv, [jnp.zeros((16,), jnp.int32) + r])
          for f in range(FD // 16):
            rv[r, pl.ds(f * 16, 16)] = rv[r, pl.ds(f * 16, 16)] * ws

      pltpu.async_copy(rows_v.at[bi], acc_sh.at[di_v.at[jnp.int32(b)]], ssem,
                       add=True)

  scatter_wait(3, 1)
  plsc.subcore_barrier()
  pltpu.sync_copy(acc_sh.at[pl.ds(s * 640, 640), :],
                  acc_hbm.at[c, pl.ds(s * 640, 640), :])


@functools.partial(
    pl.kernel,
    out_type=jax.ShapeDtypeStruct((IDX_PAD, FD), jnp.float32),
    mesh=_mesh,
    compiler_params=pltpu.CompilerParams(needs_layout_passes=False),
    scratch_types=[
        pltpu.VMEM((IPW,), jnp.int32),
        pltpu.VMEM((IPW, FD), jnp.float32),
        pltpu.VMEM((IPW, FD), jnp.float32),
        pltpu.VMEM((IPW, FD), jnp.float32),
        pltpu.SemaphoreType.DMA,
        pltpu.SemaphoreType.DMA,
        pltpu.SemaphoreType.DMA,
    ],
)
def _final_gather_kernel(r2_hbm, acc_hbm, idx_hbm, out_hbm,
                         idx_v, a_v, b_v, c_v, sem0, sem1, sem2):
  w = _wid()
  base = w * IPW
  pltpu.sync_copy(idx_hbm.at[pl.ds(base, IPW)], idx_v)
  cp0 = pltpu.async_copy(r2_hbm.at[idx_v], a_v, sem0)
  cp1 = pltpu.async_copy(acc_hbm.at[jnp.int32(0)].at[idx_v], b_v, sem1)
  cp2 = pltpu.async_copy(acc_hbm.at[jnp.int32(1)].at[idx_v], c_v, sem2)
  cp0.wait()
  cp1.wait()
  cp2.wait()

  @_loop32(IPW)
  def _(r):
    for f in range(FD // 16):
      sl = pl.ds(f * 16, 16)
      a_v[r, sl] = a_v[r, sl] + b_v[r, sl] + c_v[r, sl]

  pltpu.sync_copy(a_v, out_hbm.at[pl.ds(base, IPW)])


_BLK = 400
_GRID = N // _BLK


def _enc_body(vf_ref, tf_ref, w1_ref, b1_ref, w2_ref, b2_ref, wc_ref, bc_ref,
              out_ref):
  hp = jax.lax.Precision.HIGHEST
  v = jnp.dot(vf_ref[...], w1_ref[...], precision=hp) + b1_ref[...]
  t = jnp.dot(tf_ref[...], w2_ref[...], precision=hp) + b2_ref[...]
  z = (jnp.dot(v, wc_ref[0:FD, :], precision=hp)
       + jnp.dot(t, wc_ref[FD:2 * FD, :], precision=hp) + bc_ref[...])
  out_ref[...] = jnp.where(z >= 0, z, 0.01 * z)


def _encoder(vf, tf, W1, b1, W2, b2, Wc, bc):
  return pl.pallas_call(
      _enc_body,
      grid=(_GRID,),
      in_specs=[
          pl.BlockSpec((_BLK, 8), lambda i: (i, _Z)),
          pl.BlockSpec((_BLK, 768), lambda i: (i, _Z)),
          pl.BlockSpec((8, FD), lambda i: (_Z, _Z)),
          pl.BlockSpec((1, FD), lambda i: (_Z, _Z)),
          pl.BlockSpec((768, FD), lambda i: (_Z, _Z)),
          pl.BlockSpec((1, FD), lambda i: (_Z, _Z)),
          pl.BlockSpec((2 * FD, FD), lambda i: (_Z, _Z)),
          pl.BlockSpec((1, FD), lambda i: (_Z, _Z)),
      ],
      out_specs=pl.BlockSpec((_BLK, FD), lambda i: (i, _Z)),
      out_shape=jax.ShapeDtypeStruct((N, FD), jnp.float32),
  )(vf, tf, W1, b1, W2, b2, Wc, bc)


def _layer_body(n_acc, x_ref, acc_ref, wrel_ref, wroot_ref, b_ref,
                y_ref, r_ref):
  hp = jax.lax.Precision.HIGHEST
  x = x_ref[...]
  if n_acc:
    x = x + acc_ref[0] + acc_ref[1]
  for r in range(NUM_REL):
    y_ref[r] = jnp.dot(x, wrel_ref[r], precision=hp)
  r_ref[...] = jnp.dot(x, wroot_ref[...], precision=hp) + b_ref[...]


def _layer_dense(x, acc, Wrel, Wroot, b):
  n_acc = acc is not None
  in_specs = [pl.BlockSpec((_BLK, FD), lambda i: (i, _Z))]
  args = [x]
  if n_acc:
    in_specs.append(pl.BlockSpec((NC, _BLK, FD), lambda i: (_Z, i, _Z)))
    args.append(acc)
  else:
    in_specs.append(pl.BlockSpec((1, 1), lambda i: (_Z, _Z)))
    args.append(jnp.zeros((1, 1), jnp.float32))
  in_specs += [
      pl.BlockSpec((NUM_REL, FD, FD), lambda i: (_Z, _Z, _Z)),
      pl.BlockSpec((FD, FD), lambda i: (_Z, _Z)),
      pl.BlockSpec((1, FD), lambda i: (_Z, _Z)),
  ]
  return pl.pallas_call(
      functools.partial(_layer_body, n_acc),
      grid=(_GRID,),
      in_specs=in_specs,
      out_specs=[
          pl.BlockSpec((NUM_REL, _BLK, FD), lambda i: (_Z, i, _Z)),
          pl.BlockSpec((_BLK, FD), lambda i: (i, _Z)),
      ],
      out_shape=[
          jax.ShapeDtypeStruct((NUM_REL, N, FD), jnp.float32),
          jax.ShapeDtypeStruct((N, FD), jnp.float32),
      ],
  )(*args, Wrel, Wroot, b)


def _head_body(g_ref, w3_ref, b3_ref, out_ref):
  out_ref[...] = (jnp.dot(g_ref[...], w3_ref[...],
                          precision=jax.lax.Precision.HIGHEST) + b3_ref[...])


def _head(g, W3, b3):
  cl = W3.shape[1]
  return pl.pallas_call(
      _head_body,
      grid=(1,),
      in_specs=[
          pl.BlockSpec((IDX_PAD, FD), lambda i: (_Z, _Z)),
          pl.BlockSpec((FD, cl), lambda i: (_Z, _Z)),
          pl.BlockSpec((1, cl), lambda i: (_Z, _Z)),
      ],
      out_specs=pl.BlockSpec((IDX_PAD, cl), lambda i: (_Z, _Z)),
      out_shape=jax.ShapeDtypeStruct((IDX_PAD, cl), jnp.float32),
  )(g, W3, b3)


def kernel(value_feature, text_feature, edge_index, edge_type, idx,
           W1, b1, W2, b2, Wc, bc,
           Wrel1, Wroot1, brg1, Wrel2, Wroot2, brg2, W3, b3):
  src = edge_index[0].astype(jnp.int32)
  dst = edge_index[1].astype(jnp.int32)
  et = edge_type.astype(jnp.int32)

  pad = E_PAD - E
  pr = jnp.arange(pad, dtype=jnp.int32)
  gidx = jnp.concatenate([et * N + src, pr % (3 * N)])
  didx = jnp.concatenate([dst, N + pr % (ACC_ROWS - N)])
  cidx = jnp.concatenate([et * N + dst, 3 * N + pr % (CNT_PAD - 3 * N)])
  idxp = jnp.concatenate(
      [idx.astype(jnp.int32), jnp.zeros((IDX_PAD - IDX,), jnp.int32)])

  b1r = b1.reshape(1, FD)
  b2r = b2.reshape(1, FD)
  bcr = bc.reshape(1, FD)
  brg1r = brg1.reshape(1, FD)
  brg2r = brg2.reshape(1, FD)
  b3r = b3.reshape(1, -1)

  cnt2 = _count_kernel(cidx)
  rc = _rc_table(cnt2.reshape(NC, CNT_PAD // 128, 128)).reshape(CNT_PAD)

  f0 = _encoder(value_feature, text_feature, W1, b1r, W2, b2r, Wc, bcr)

  y1, r1 = _layer_dense(f0, None, Wrel1, Wroot1, brg1r)
  acc1 = _edge_kernel(y1.reshape(NUM_REL * N, FD), gidx, didx, cidx, rc)

  y2, r2 = _layer_dense(r1, acc1, Wrel2, Wroot2, brg2r)
  acc2 = _edge_kernel(y2.reshape(NUM_REL * N, FD), gidx, didx, cidx, rc)

  g = _final_gather_kernel(r2, acc2, idxp)
  out = _head(g, W3, b3r)
  return out[:IDX]


# speedup vs baseline: 2.6768x; 1.0478x over previous
import functools
import numpy as np
import jax
import jax.numpy as jnp
from jax import lax
from jax.experimental import pallas as pl
from jax.experimental.pallas import tpu as pltpu
from jax.experimental.pallas import tpu_sc as plsc

N = 10000
E = 320000
FD = 128
NUM_REL = 3
IDX = 2000

NC = 2
NS = 16
NW = NC * NS

CH = 128
EPW = 10240
E_PAD = EPW * NW
N_CHUNKS = EPW // CH

CNT_PAD = 30720
ACC_ROWS = 10240
IDX_PAD = 2048
IPW = IDX_PAD // NW

_Z = np.int32(0)

_mesh = plsc.VectorSubcoreMesh(core_axis_name="c", subcore_axis_name="s")


def _wid():
  return lax.axis_index("c") * NS + lax.axis_index("s")


def _loop32(n, unroll=None):
  del unroll
  return pl.loop(jnp.int32(0), jnp.int32(n))


def _zero_fill(buf, n16):
  z = jnp.zeros((16,), jnp.float32)
  if buf.ndim == 1:
    @_loop32(n16)
    def _(i):
      buf[pl.ds(i * 16, 16)] = z
  else:
    rows, cols = buf.shape

    @_loop32(rows)
    def _(i):
      for f in range(cols // 16):
        buf[i, pl.ds(f * 16, 16)] = z


@functools.partial(
    pl.kernel,
    out_type=jax.ShapeDtypeStruct((NC, CNT_PAD), jnp.float32),
    mesh=_mesh,
    compiler_params=pltpu.CompilerParams(needs_layout_passes=False),
    scratch_types=[
        pltpu.MemorySpace.VMEM_SHARED((CNT_PAD,), jnp.float32),
        pltpu.VMEM((2, CH), jnp.int32),
        pltpu.VMEM((CH,), jnp.float32),
        pltpu.VMEM((1920,), jnp.float32),
        [pltpu.SemaphoreType.DMA] * 2,
        pltpu.SemaphoreType.DMA,
    ],
)
def _count_kernel(cidx_hbm, cnt2_hbm, cnt_sh, idxb, ones_v, zbuf, lsem, ssem):
  c = lax.axis_index("c")
  s = lax.axis_index("s")
  _zero_fill(zbuf, 120)
  pltpu.sync_copy(zbuf, cnt_sh.at[pl.ds(s * 1920, 1920)])
  one = jnp.ones((16,), jnp.float32)
  for g in range(CH // 16):
    ones_v[pl.ds(g * 16, 16)] = one
  plsc.subcore_barrier()

  w = _wid()
  base_w = w * EPW

  def bload(slot, j):
    pltpu.async_copy(cidx_hbm.at[pl.ds(base_w + j * CH, CH)],
                     idxb.at[jnp.int32(slot)], lsem[slot])

  def bload_wait(slot):
    pltpu.make_async_copy(cidx_hbm.at[pl.ds(base_w, CH)],
                          idxb.at[jnp.int32(slot)], lsem[slot]).wait()

  def bscat_wait(slot):
    pltpu.make_async_copy(ones_v, cnt_sh.at[idxb.at[jnp.int32(slot)]],
                          ssem).wait()

  bload(0, jnp.int32(0))

  @_loop32(N_CHUNKS // 2)
  def _(j2):
    for b in (0, 1):
      j = j2 * 2 + b
      bload_wait(b)

      @pl.when(j >= 1)
      def _():
        bscat_wait(1 - b)

      @pl.when(j + 1 < N_CHUNKS)
      def _():
        bload(1 - b, j + 1)

      pltpu.async_copy(ones_v, cnt_sh.at[idxb.at[jnp.int32(b)]], ssem,
                       add=True)

  bscat_wait(1)
  plsc.subcore_barrier()
  pltpu.sync_copy(cnt_sh.at[pl.ds(s * 1920, 1920)],
                  cnt2_hbm.at[c, pl.ds(s * 1920, 1920)])


def _rc_body(cnt_ref, rc_ref):
  t = cnt_ref[0] + cnt_ref[1]
  rc_ref[...] = 1.0 / jnp.maximum(t, 1.0)


def _rc_table(cnt2):
  return pl.pallas_call(
      _rc_body,
      grid=(1,),
      in_specs=[pl.BlockSpec((NC, CNT_PAD // 128, 128),
                             lambda i: (_Z, _Z, _Z))],
      out_specs=pl.BlockSpec((CNT_PAD // 128, 128), lambda i: (_Z, _Z)),
      out_shape=jax.ShapeDtypeStruct((CNT_PAD // 128, 128), jnp.float32),
  )(cnt2)


@functools.partial(
    pl.kernel,
    out_type=jax.ShapeDtypeStruct((NC, ACC_ROWS, FD), jnp.float32),
    mesh=_mesh,
    compiler_params=pltpu.CompilerParams(needs_layout_passes=False),
    scratch_types=[
        pltpu.MemorySpace.VMEM_SHARED((ACC_ROWS, FD), jnp.float32),
        pltpu.VMEM((4, CH), jnp.int32),
        pltpu.VMEM((4, CH), jnp.int32),
        pltpu.VMEM((4, CH), jnp.int32),
        pltpu.VMEM((4, CH), jnp.float32),
        pltpu.VMEM((2, CH, FD), jnp.float32),
        pltpu.VMEM((16, FD), jnp.float32),
        [pltpu.SemaphoreType.DMA] * 4,
        [pltpu.SemaphoreType.DMA] * 4,
        [pltpu.SemaphoreType.DMA] * 2,
        pltpu.SemaphoreType.DMA,
    ],
)
def _edge_kernel(y_hbm, gidx_hbm, didx_hbm, cidx_hbm, rc_hbm, acc_hbm,
                 acc_sh, gi_v, di_v, ci_v, w_v, rows_v, zbuf,
                 isem, wsem, gsem, ssem):
  c = lax.axis_index("c")
  s = lax.axis_index("s")
  _zero_fill(zbuf, 128)

  @_loop32(40)
  def _(k):
    pltpu.sync_copy(zbuf, acc_sh.at[pl.ds(s * 640 + k * 16, 16), :])

  plsc.subcore_barrier()

  w = _wid()
  base_w = w * EPW

  def idx_load(s4, j):
    si = jnp.int32(s4)
    b = base_w + j * CH
    pltpu.async_copy(gidx_hbm.at[pl.ds(b, CH)], gi_v.at[si], isem[s4])
    pltpu.async_copy(didx_hbm.at[pl.ds(b, CH)], di_v.at[si], isem[s4])
    pltpu.async_copy(cidx_hbm.at[pl.ds(b, CH)], ci_v.at[si], isem[s4])

  def idx_wait(s4):
    si = jnp.int32(s4)
    b0 = pl.ds(base_w, CH)
    pltpu.make_async_copy(gidx_hbm.at[b0], gi_v.at[si], isem[s4]).wait()
    pltpu.make_async_copy(didx_hbm.at[b0], di_v.at[si], isem[s4]).wait()
    pltpu.make_async_copy(cidx_hbm.at[b0], ci_v.at[si], isem[s4]).wait()

  def wg_start(s4):
    si = jnp.int32(s4)
    pltpu.async_copy(rc_hbm.at[ci_v.at[si]], w_v.at[si], wsem[s4])

  def wg_wait(s4):
    si = jnp.int32(s4)
    pltpu.make_async_copy(rc_hbm.at[ci_v.at[si]], w_v.at[si],
                          wsem[s4]).wait()

  def gather_start(s4, s2):
    pltpu.async_copy(y_hbm.at[gi_v.at[jnp.int32(s4)]],
                     rows_v.at[jnp.int32(s2)], gsem[s2])

  def gather_wait(s4, s2):
    pltpu.make_async_copy(y_hbm.at[gi_v.at[jnp.int32(s4)]],
                          rows_v.at[jnp.int32(s2)], gsem[s2]).wait()

  def scatter_wait(s4, s2):
    pltpu.make_async_copy(rows_v.at[jnp.int32(s2)],
                          acc_sh.at[di_v.at[jnp.int32(s4)]], ssem).wait()

  idx_load(0, jnp.int32(0))
  idx_load(1, jnp.int32(1))
  idx_wait(0)
  wg_start(0)
  gather_start(0, 0)

  @_loop32(N_CHUNKS // 4)
  def _(j4):
    for b in range(4):
      j = j4 * 4 + b
      s2 = b % 2
      gather_wait(b, s2)

      @pl.when(j + 2 < N_CHUNKS)
      def _():
        idx_load((b + 2) % 4, j + 2)

      @pl.when(j + 1 < N_CHUNKS)
      def _():
        idx_wait((b + 1) % 4)
        wg_start((b + 1) % 4)

      @pl.when(j >= 1)
      def _():
        scatter_wait((b - 1) % 4, 1 - s2)

      @pl.when(j + 1 < N_CHUNKS)
      def _():
        gather_start((b + 1) % 4, 1 - s2)

      wg_wait(b)
      bi = jnp.int32(s2)
      rv = rows_v.at[bi]
      wv = w_v.at[jnp.int32(b)]

      @_loop32(CH // 8)
      def _(r8):
        for u in range(8):
          r = r8 * 8 + u
          ws = plsc.load_gather(wv, [jnp.zeros((16,), jnp.int32) + r])
          for f in range(FD // 16):
            rv[r, pl.ds(f * 16, 16)] = rv[r, pl.ds(f * 16, 16)] * ws

      pltpu.async_copy(rows_v.at[bi], acc_sh.at[di_v.at[jnp.int32(b)]], ssem,
                       add=True)

  scatter_wait(3, 1)
  plsc.subcore_barrier()
  pltpu.sync_copy(acc_sh.at[pl.ds(s * 640, 640), :],
                  acc_hbm.at[c, pl.ds(s * 640, 640), :])


@functools.partial(
    pl.kernel,
    out_type=jax.ShapeDtypeStruct((IDX_PAD, FD), jnp.float32),
    mesh=_mesh,
    compiler_params=pltpu.CompilerParams(needs_layout_passes=False),
    scratch_types=[
        pltpu.VMEM((IPW,), jnp.int32),
        pltpu.VMEM((IPW, FD), jnp.float32),
        pltpu.VMEM((IPW, FD), jnp.float32),
        pltpu.VMEM((IPW, FD), jnp.float32),
        pltpu.SemaphoreType.DMA,
        pltpu.SemaphoreType.DMA,
        pltpu.SemaphoreType.DMA,
    ],
)
def _final_gather_kernel(r2_hbm, acc_hbm, idx_hbm, out_hbm,
                         idx_v, a_v, b_v, c_v, sem0, sem1, sem2):
  w = _wid()
  base = w * IPW
  pltpu.sync_copy(idx_hbm.at[pl.ds(base, IPW)], idx_v)
  cp0 = pltpu.async_copy(r2_hbm.at[idx_v], a_v, sem0)
  cp1 = pltpu.async_copy(acc_hbm.at[jnp.int32(0)].at[idx_v], b_v, sem1)
  cp2 = pltpu.async_copy(acc_hbm.at[jnp.int32(1)].at[idx_v], c_v, sem2)
  cp0.wait()
  cp1.wait()
  cp2.wait()

  @_loop32(IPW)
  def _(r):
    for f in range(FD // 16):
      sl = pl.ds(f * 16, 16)
      a_v[r, sl] = a_v[r, sl] + b_v[r, sl] + c_v[r, sl]

  pltpu.sync_copy(a_v, out_hbm.at[pl.ds(base, IPW)])


_BLK = 400
_GRID = N // _BLK


def _enc_l1_body(vf_ref, tf_ref, w1_ref, b1_ref, w2_ref, b2_ref, wc_ref,
                bc_ref, wrel_ref, wroot_ref, brg_ref, cnt_ref,
                y_ref, r_ref, rc_ref):
  hp = None
  v = jnp.dot(vf_ref[...], w1_ref[...], precision=hp) + b1_ref[...]
  t = jnp.dot(tf_ref[...], w2_ref[...], precision=hp) + b2_ref[...]
  z = (jnp.dot(v, wc_ref[0:FD, :], precision=hp)
       + jnp.dot(t, wc_ref[FD:2 * FD, :], precision=hp) + bc_ref[...])
  f0 = jnp.where(z >= 0, z, 0.01 * z)
  for r in range(NUM_REL):
    y_ref[r] = jnp.dot(f0, wrel_ref[r], precision=hp)
  r_ref[...] = jnp.dot(f0, wroot_ref[...], precision=hp) + brg_ref[...]

  @pl.when(pl.program_id(0) == 0)
  def _():
    rc_ref[...] = 1.0 / jnp.maximum(cnt_ref[0] + cnt_ref[1], 1.0)


def _enc_l1(vf, tf, W1, b1, W2, b2, Wc, bc, Wrel, Wroot, brg, cnt2):
  return pl.pallas_call(
      _enc_l1_body,
      grid=(_GRID,),
      in_specs=[
          pl.BlockSpec((_BLK, 8), lambda i: (i, _Z)),
          pl.BlockSpec((_BLK, 768), lambda i: (i, _Z)),
          pl.BlockSpec((8, FD), lambda i: (_Z, _Z)),
          pl.BlockSpec((1, FD), lambda i: (_Z, _Z)),
          pl.BlockSpec((768, FD), lambda i: (_Z, _Z)),
          pl.BlockSpec((1, FD), lambda i: (_Z, _Z)),
          pl.BlockSpec((2 * FD, FD), lambda i: (_Z, _Z)),
          pl.BlockSpec((1, FD), lambda i: (_Z, _Z)),
          pl.BlockSpec((NUM_REL, FD, FD), lambda i: (_Z, _Z, _Z)),
          pl.BlockSpec((FD, FD), lambda i: (_Z, _Z)),
          pl.BlockSpec((1, FD), lambda i: (_Z, _Z)),
          pl.BlockSpec((NC, CNT_PAD // 128, 128), lambda i: (_Z, _Z, _Z)),
      ],
      out_specs=[
          pl.BlockSpec((NUM_REL, _BLK, FD), lambda i: (_Z, i, _Z)),
          pl.BlockSpec((_BLK, FD), lambda i: (i, _Z)),
          pl.BlockSpec((CNT_PAD // 128, 128), lambda i: (_Z, _Z)),
      ],
      out_shape=[
          jax.ShapeDtypeStruct((NUM_REL, N, FD), jnp.float32),
          jax.ShapeDtypeStruct((N, FD), jnp.float32),
          jax.ShapeDtypeStruct((CNT_PAD // 128, 128), jnp.float32),
      ],
  )(vf, tf, W1, b1, W2, b2, Wc, bc, Wrel, Wroot, brg, cnt2)


def _layer_body(n_acc, x_ref, acc_ref, wrel_ref, wroot_ref, b_ref,
                y_ref, r_ref):
  hp = None
  x = x_ref[...]
  if n_acc:
    x = x + acc_ref[0] + acc_ref[1]
  for r in range(NUM_REL):
    y_ref[r] = jnp.dot(x, wrel_ref[r], precision=hp)
  r_ref[...] = jnp.dot(x, wroot_ref[...], precision=hp) + b_ref[...]


def _layer_dense(x, acc, Wrel, Wroot, b):
  n_acc = acc is not None
  in_specs = [pl.BlockSpec((_BLK, FD), lambda i: (i, _Z))]
  args = [x]
  if n_acc:
    in_specs.append(pl.BlockSpec((NC, _BLK, FD), lambda i: (_Z, i, _Z)))
    args.append(acc)
  else:
    in_specs.append(pl.BlockSpec((1, 1), lambda i: (_Z, _Z)))
    args.append(jnp.zeros((1, 1), jnp.float32))
  in_specs += [
      pl.BlockSpec((NUM_REL, FD, FD), lambda i: (_Z, _Z, _Z)),
      pl.BlockSpec((FD, FD), lambda i: (_Z, _Z)),
      pl.BlockSpec((1, FD), lambda i: (_Z, _Z)),
  ]
  return pl.pallas_call(
      functools.partial(_layer_body, n_acc),
      grid=(_GRID,),
      in_specs=in_specs,
      out_specs=[
          pl.BlockSpec((NUM_REL, _BLK, FD), lambda i: (_Z, i, _Z)),
          pl.BlockSpec((_BLK, FD), lambda i: (i, _Z)),
      ],
      out_shape=[
          jax.ShapeDtypeStruct((NUM_REL, N, FD), jnp.float32),
          jax.ShapeDtypeStruct((N, FD), jnp.float32),
      ],
  )(*args, Wrel, Wroot, b)


def _head_body(g_ref, w3_ref, b3_ref, out_ref):
  out_ref[...] = jnp.dot(g_ref[...], w3_ref[...]) + b3_ref[...]


def _head(g, W3, b3):
  cl = W3.shape[1]
  return pl.pallas_call(
      _head_body,
      grid=(1,),
      in_specs=[
          pl.BlockSpec((IDX_PAD, FD), lambda i: (_Z, _Z)),
          pl.BlockSpec((FD, cl), lambda i: (_Z, _Z)),
          pl.BlockSpec((1, cl), lambda i: (_Z, _Z)),
      ],
      out_specs=pl.BlockSpec((IDX_PAD, cl), lambda i: (_Z, _Z)),
      out_shape=jax.ShapeDtypeStruct((IDX_PAD, cl), jnp.float32),
  )(g, W3, b3)


def kernel(value_feature, text_feature, edge_index, edge_type, idx,
           W1, b1, W2, b2, Wc, bc,
           Wrel1, Wroot1, brg1, Wrel2, Wroot2, brg2, W3, b3):
  src = edge_index[0].astype(jnp.int32)
  dst = edge_index[1].astype(jnp.int32)
  et = edge_type.astype(jnp.int32)

  pad = E_PAD - E
  pr = jnp.arange(pad, dtype=jnp.int32)
  gidx = jnp.concatenate([et * N + src, pr % (3 * N)])
  didx = jnp.concatenate([dst, N + pr % (ACC_ROWS - N)])
  cidx = jnp.concatenate([et * N + dst, 3 * N + pr % (CNT_PAD - 3 * N)])
  idxp = jnp.concatenate(
      [idx.astype(jnp.int32), jnp.zeros((IDX_PAD - IDX,), jnp.int32)])

  b1r = b1.reshape(1, FD)
  b2r = b2.reshape(1, FD)
  bcr = bc.reshape(1, FD)
  brg1r = brg1.reshape(1, FD)
  brg2r = brg2.reshape(1, FD)
  b3r = b3.reshape(1, -1)

  cnt2 = _count_kernel(cidx)
  y1, r1, rc2 = _enc_l1(value_feature, text_feature, W1, b1r, W2, b2r,
                        Wc, bcr, Wrel1, Wroot1, brg1r,
                        cnt2.reshape(NC, CNT_PAD // 128, 128))
  rc = rc2.reshape(CNT_PAD)
  acc1 = _edge_kernel(y1.reshape(NUM_REL * N, FD), gidx, didx, cidx, rc)

  y2, r2 = _layer_dense(r1, acc1, Wrel2, Wroot2, brg2r)
  acc2 = _edge_kernel(y2.reshape(NUM_REL * N, FD), gidx, didx, cidx, rc)

  g = _final_gather_kernel(r2, acc2, idxp)
  out = _head(g, W3, b3r)
  return out[:IDX]


# speedup vs baseline: 2.6809x; 1.0015x over previous
import functools
import numpy as np
import jax
import jax.numpy as jnp
from jax import lax
from jax.experimental import pallas as pl
from jax.experimental.pallas import tpu as pltpu
from jax.experimental.pallas import tpu_sc as plsc

N = 10000
E = 320000
FD = 128
NUM_REL = 3
IDX = 2000

NC = 2
NS = 16
NW = NC * NS

CH = 128
EPW = 10240
E_PAD = EPW * NW
N_CHUNKS = EPW // CH

CNT_PAD = 30720
ACC_ROWS = 10240
IDX_PAD = 2048
IPW = IDX_PAD // NW

_Z = np.int32(0)

_mesh = plsc.VectorSubcoreMesh(core_axis_name="c", subcore_axis_name="s")


def _wid():
  return lax.axis_index("c") * NS + lax.axis_index("s")


def _loop32(n, unroll=None):
  del unroll
  return pl.loop(jnp.int32(0), jnp.int32(n))


def _zero_fill(buf, n16):
  z = jnp.zeros((16,), jnp.float32)
  if buf.ndim == 1:
    @_loop32(n16)
    def _(i):
      buf[pl.ds(i * 16, 16)] = z
  else:
    rows, cols = buf.shape

    @_loop32(rows)
    def _(i):
      for f in range(cols // 16):
        buf[i, pl.ds(f * 16, 16)] = z


@functools.partial(
    pl.kernel,
    out_type=jax.ShapeDtypeStruct((NC, CNT_PAD), jnp.float32),
    mesh=_mesh,
    compiler_params=pltpu.CompilerParams(needs_layout_passes=False),
    scratch_types=[
        pltpu.MemorySpace.VMEM_SHARED((CNT_PAD,), jnp.float32),
        pltpu.VMEM((2, CH), jnp.int32),
        pltpu.VMEM((CH,), jnp.float32),
        pltpu.VMEM((1920,), jnp.float32),
        [pltpu.SemaphoreType.DMA] * 2,
        pltpu.SemaphoreType.DMA,
    ],
)
def _count_kernel(cidx_hbm, cnt2_hbm, cnt_sh, idxb, ones_v, zbuf, lsem, ssem):
  c = lax.axis_index("c")
  s = lax.axis_index("s")
  _zero_fill(zbuf, 120)
  pltpu.sync_copy(zbuf, cnt_sh.at[pl.ds(s * 1920, 1920)])
  one = jnp.ones((16,), jnp.float32)
  for g in range(CH // 16):
    ones_v[pl.ds(g * 16, 16)] = one
  plsc.subcore_barrier()

  w = _wid()
  base_w = w * EPW

  def bload(slot, j):
    pltpu.async_copy(cidx_hbm.at[pl.ds(base_w + j * CH, CH)],
                     idxb.at[jnp.int32(slot)], lsem[slot])

  def bload_wait(slot):
    pltpu.make_async_copy(cidx_hbm.at[pl.ds(base_w, CH)],
                          idxb.at[jnp.int32(slot)], lsem[slot]).wait()

  def bscat_wait(slot):
    pltpu.make_async_copy(ones_v, cnt_sh.at[idxb.at[jnp.int32(slot)]],
                          ssem).wait()

  bload(0, jnp.int32(0))

  @_loop32(N_CHUNKS // 2)
  def _(j2):
    for b in (0, 1):
      j = j2 * 2 + b
      bload_wait(b)

      @pl.when(j >= 1)
      def _():
        bscat_wait(1 - b)

      @pl.when(j + 1 < N_CHUNKS)
      def _():
        bload(1 - b, j + 1)

      pltpu.async_copy(ones_v, cnt_sh.at[idxb.at[jnp.int32(b)]], ssem,
                       add=True)

  bscat_wait(1)
  plsc.subcore_barrier()
  pltpu.sync_copy(cnt_sh.at[pl.ds(s * 1920, 1920)],
                  cnt2_hbm.at[c, pl.ds(s * 1920, 1920)])


@functools.partial(
    pl.kernel,
    out_type=jax.ShapeDtypeStruct((NC, ACC_ROWS, FD), jnp.float32),
    mesh=_mesh,
    compiler_params=pltpu.CompilerParams(needs_layout_passes=False),
    scratch_types=[
        pltpu.MemorySpace.VMEM_SHARED((ACC_ROWS, FD), jnp.float32),
        pltpu.VMEM((4, CH), jnp.int32),
        pltpu.VMEM((4, CH), jnp.int32),
        pltpu.VMEM((4, CH), jnp.int32),
        pltpu.VMEM((4, CH), jnp.float32),
        pltpu.VMEM((2, CH, FD), jnp.float32),
        pltpu.VMEM((16, FD), jnp.float32),
        [pltpu.SemaphoreType.DMA] * 4,
        [pltpu.SemaphoreType.DMA] * 4,
        [pltpu.SemaphoreType.DMA] * 2,
        pltpu.SemaphoreType.DMA,
    ],
)
def _edge_kernel(y_hbm, gidx_hbm, didx_hbm, cidx_hbm, rc_hbm, acc_hbm,
                 acc_sh, gi_v, di_v, ci_v, w_v, rows_v, zbuf,
                 isem, wsem, gsem, ssem):
  c = lax.axis_index("c")
  s = lax.axis_index("s")
  _zero_fill(zbuf, 128)

  @_loop32(40)
  def _(k):
    pltpu.sync_copy(zbuf, acc_sh.at[pl.ds(s * 640 + k * 16, 16), :])

  plsc.subcore_barrier()

  w = _wid()
  base_w = w * EPW

  def idx_load(s4, j):
    si = jnp.int32(s4)
    b = base_w + j * CH
    pltpu.async_copy(gidx_hbm.at[pl.ds(b, CH)], gi_v.at[si], isem[s4])
    pltpu.async_copy(didx_hbm.at[pl.ds(b, CH)], di_v.at[si], isem[s4])
    pltpu.async_copy(cidx_hbm.at[pl.ds(b, CH)], ci_v.at[si], isem[s4])

  def idx_wait(s4):
    si = jnp.int32(s4)
    b0 = pl.ds(base_w, CH)
    pltpu.make_async_copy(gidx_hbm.at[b0], gi_v.at[si], isem[s4]).wait()
    pltpu.make_async_copy(didx_hbm.at[b0], di_v.at[si], isem[s4]).wait()
    pltpu.make_async_copy(cidx_hbm.at[b0], ci_v.at[si], isem[s4]).wait()

  def wg_start(s4):
    si = jnp.int32(s4)
    pltpu.async_copy(rc_hbm.at[ci_v.at[si]], w_v.at[si], wsem[s4])

  def wg_wait(s4):
    si = jnp.int32(s4)
    pltpu.make_async_copy(rc_hbm.at[ci_v.at[si]], w_v.at[si],
                          wsem[s4]).wait()

  def gather_start(s4, s2):
    pltpu.async_copy(y_hbm.at[gi_v.at[jnp.int32(s4)]],
                     rows_v.at[jnp.int32(s2)], gsem[s2])

  def gather_wait(s4, s2):
    pltpu.make_async_copy(y_hbm.at[gi_v.at[jnp.int32(s4)]],
                          rows_v.at[jnp.int32(s2)], gsem[s2]).wait()

  def scatter_wait(s4, s2):
    pltpu.make_async_copy(rows_v.at[jnp.int32(s2)],
                          acc_sh.at[di_v.at[jnp.int32(s4)]], ssem).wait()

  idx_load(0, jnp.int32(0))
  idx_load(1, jnp.int32(1))
  idx_wait(0)
  wg_start(0)
  gather_start(0, 0)

  @_loop32(N_CHUNKS // 4)
  def _(j4):
    for b in range(4):
      j = j4 * 4 + b
      s2 = b % 2
      gather_wait(b, s2)

      @pl.when(j + 2 < N_CHUNKS)
      def _():
        idx_load((b + 2) % 4, j + 2)

      @pl.when(j + 1 < N_CHUNKS)
      def _():
        idx_wait((b + 1) % 4)
        wg_start((b + 1) % 4)

      @pl.when(j >= 1)
      def _():
        scatter_wait((b - 1) % 4, 1 - s2)

      @pl.when(j + 1 < N_CHUNKS)
      def _():
        gather_start((b + 1) % 4, 1 - s2)

      wg_wait(b)
      bi = jnp.int32(s2)
      rv = rows_v.at[bi]
      wv = w_v.at[jnp.int32(b)]

      @_loop32(CH // 8)
      def _(r8):
        for u in range(8):
          r = r8 * 8 + u
          ws = plsc.load_gather(wv, [jnp.zeros((16,), jnp.int32) + r])
          for f in range(FD // 16):
            rv[r, pl.ds(f * 16, 16)] = rv[r, pl.ds(f * 16, 16)] * ws

      pltpu.async_copy(rows_v.at[bi], acc_sh.at[di_v.at[jnp.int32(b)]], ssem,
                       add=True)

  scatter_wait(3, 1)
  plsc.subcore_barrier()
  pltpu.sync_copy(acc_sh.at[pl.ds(s * 640, 640), :],
                  acc_hbm.at[c, pl.ds(s * 640, 640), :])


@functools.partial(
    pl.kernel,
    out_type=jax.ShapeDtypeStruct((IDX_PAD, FD), jnp.float32),
    mesh=_mesh,
    compiler_params=pltpu.CompilerParams(needs_layout_passes=False),
    scratch_types=[
        pltpu.VMEM((IPW,), jnp.int32),
        pltpu.VMEM((IPW, FD), jnp.float32),
        pltpu.VMEM((IPW, FD), jnp.float32),
        pltpu.VMEM((IPW, FD), jnp.float32),
        pltpu.SemaphoreType.DMA,
        pltpu.SemaphoreType.DMA,
        pltpu.SemaphoreType.DMA,
    ],
)
def _final_gather_kernel(r2_hbm, acc_hbm, idx_hbm, out_hbm,
                         idx_v, a_v, b_v, c_v, sem0, sem1, sem2):
  w = _wid()
  base = w * IPW
  pltpu.sync_copy(idx_hbm.at[pl.ds(base, IPW)], idx_v)
  cp0 = pltpu.async_copy(r2_hbm.at[idx_v], a_v, sem0)
  cp1 = pltpu.async_copy(acc_hbm.at[jnp.int32(0)].at[idx_v], b_v, sem1)
  cp2 = pltpu.async_copy(acc_hbm.at[jnp.int32(1)].at[idx_v], c_v, sem2)
  cp0.wait()
  cp1.wait()
  cp2.wait()

  @_loop32(IPW)
  def _(r):
    for f in range(FD // 16):
      sl = pl.ds(f * 16, 16)
      a_v[r, sl] = a_v[r, sl] + b_v[r, sl] + c_v[r, sl]

  pltpu.sync_copy(a_v, out_hbm.at[pl.ds(base, IPW)])


_BLK = 400
_GRID = N // _BLK


def _enc_l1_body(vf_ref, tf_ref, w1_ref, b1_ref, w2_ref, b2_ref, wc_ref,
                bc_ref, wrel_ref, wroot_ref, brg_ref, cnt_ref,
                y_ref, r_ref, rc_ref):
  hp = None
  v = jnp.dot(vf_ref[...], w1_ref[...], precision=hp) + b1_ref[...]
  t = jnp.dot(tf_ref[...], w2_ref[...], precision=hp) + b2_ref[...]
  z = (jnp.dot(v, wc_ref[0:FD, :], precision=hp)
       + jnp.dot(t, wc_ref[FD:2 * FD, :], precision=hp) + bc_ref[...])
  f0 = jnp.where(z >= 0, z, 0.01 * z)
  for r in range(NUM_REL):
    y_ref[r] = jnp.dot(f0, wrel_ref[r], precision=hp)
  r_ref[...] = jnp.dot(f0, wroot_ref[...], precision=hp) + brg_ref[...]

  @pl.when(pl.program_id(0) == 0)
  def _():
    rc_ref[...] = 1.0 / jnp.maximum(cnt_ref[0] + cnt_ref[1], 1.0)


def _enc_l1(vf, tf, W1, b1, W2, b2, Wc, bc, Wrel, Wroot, brg, cnt2):
  return pl.pallas_call(
      _enc_l1_body,
      grid=(_GRID,),
      in_specs=[
          pl.BlockSpec((_BLK, 8), lambda i: (i, _Z)),
          pl.BlockSpec((_BLK, 768), lambda i: (i, _Z)),
          pl.BlockSpec((8, FD), lambda i: (_Z, _Z)),
          pl.BlockSpec((1, FD), lambda i: (_Z, _Z)),
          pl.BlockSpec((768, FD), lambda i: (_Z, _Z)),
          pl.BlockSpec((1, FD), lambda i: (_Z, _Z)),
          pl.BlockSpec((2 * FD, FD), lambda i: (_Z, _Z)),
          pl.BlockSpec((1, FD), lambda i: (_Z, _Z)),
          pl.BlockSpec((NUM_REL, FD, FD), lambda i: (_Z, _Z, _Z)),
          pl.BlockSpec((FD, FD), lambda i: (_Z, _Z)),
          pl.BlockSpec((1, FD), lambda i: (_Z, _Z)),
          pl.BlockSpec((NC, CNT_PAD // 128, 128), lambda i: (_Z, _Z, _Z)),
      ],
      out_specs=[
          pl.BlockSpec((NUM_REL, _BLK, FD), lambda i: (_Z, i, _Z)),
          pl.BlockSpec((_BLK, FD), lambda i: (i, _Z)),
          pl.BlockSpec((CNT_PAD // 128, 128), lambda i: (_Z, _Z)),
      ],
      out_shape=[
          jax.ShapeDtypeStruct((NUM_REL, N, FD), jnp.float32),
          jax.ShapeDtypeStruct((N, FD), jnp.float32),
          jax.ShapeDtypeStruct((CNT_PAD // 128, 128), jnp.float32),
      ],
  )(vf, tf, W1, b1, W2, b2, Wc, bc, Wrel, Wroot, brg, cnt2)


def _layer_body(x_ref, acc_ref, wrel_ref, wroot_ref, b_ref, y_ref, r_ref):
  hp = None
  x = x_ref[...] + acc_ref[0] + acc_ref[1]
  for r in range(NUM_REL):
    y_ref[r] = jnp.dot(x, wrel_ref[r], precision=hp)
  r_ref[...] = jnp.dot(x, wroot_ref[...], precision=hp) + b_ref[...]


def _layer_dense(x, acc, Wrel, Wroot, b):
  return pl.pallas_call(
      _layer_body,
      grid=(_GRID,),
      in_specs=[
          pl.BlockSpec((_BLK, FD), lambda i: (i, _Z)),
          pl.BlockSpec((NC, _BLK, FD), lambda i: (_Z, i, _Z)),
          pl.BlockSpec((NUM_REL, FD, FD), lambda i: (_Z, _Z, _Z)),
          pl.BlockSpec((FD, FD), lambda i: (_Z, _Z)),
          pl.BlockSpec((1, FD), lambda i: (_Z, _Z)),
      ],
      out_specs=[
          pl.BlockSpec((NUM_REL, _BLK, FD), lambda i: (_Z, i, _Z)),
          pl.BlockSpec((_BLK, FD), lambda i: (i, _Z)),
      ],
      out_shape=[
          jax.ShapeDtypeStruct((NUM_REL, N, FD), jnp.float32),
          jax.ShapeDtypeStruct((N, FD), jnp.float32),
      ],
  )(x, acc, Wrel, Wroot, b)


def _head_body(g_ref, w3_ref, b3_ref, out_ref):
  out_ref[...] = jnp.dot(g_ref[...], w3_ref[...]) + b3_ref[...]


def _head(g, W3, b3):
  cl = W3.shape[1]
  return pl.pallas_call(
      _head_body,
      grid=(1,),
      in_specs=[
          pl.BlockSpec((IDX_PAD, FD), lambda i: (_Z, _Z)),
          pl.BlockSpec((FD, cl), lambda i: (_Z, _Z)),
          pl.BlockSpec((1, cl), lambda i: (_Z, _Z)),
      ],
      out_specs=pl.BlockSpec((IDX_PAD, cl), lambda i: (_Z, _Z)),
      out_shape=jax.ShapeDtypeStruct((IDX_PAD, cl), jnp.float32),
  )(g, W3, b3)


def kernel(value_feature, text_feature, edge_index, edge_type, idx,
           W1, b1, W2, b2, Wc, bc,
           Wrel1, Wroot1, brg1, Wrel2, Wroot2, brg2, W3, b3):
  src = edge_index[0].astype(jnp.int32)
  dst = edge_index[1].astype(jnp.int32)
  et = edge_type.astype(jnp.int32)

  pad = E_PAD - E
  pr = jnp.arange(pad, dtype=jnp.int32)
  gidx = jnp.concatenate([et * N + src, pr % (3 * N)])
  didx = jnp.concatenate([dst, N + pr % (ACC_ROWS - N)])
  cidx = jnp.concatenate([et * N + dst, 3 * N + pr % (CNT_PAD - 3 * N)])
  idxp = jnp.concatenate(
      [idx.astype(jnp.int32), jnp.zeros((IDX_PAD - IDX,), jnp.int32)])

  b1r = b1.reshape(1, FD)
  b2r = b2.reshape(1, FD)
  bcr = bc.reshape(1, FD)
  brg1r = brg1.reshape(1, FD)
  brg2r = brg2.reshape(1, FD)
  b3r = b3.reshape(1, -1)

  cnt2 = _count_kernel(cidx)
  y1, r1, rc2 = _enc_l1(value_feature, text_feature, W1, b1r, W2, b2r,
                        Wc, bcr, Wrel1, Wroot1, brg1r,
                        cnt2.reshape(NC, CNT_PAD // 128, 128))
  rc = rc2.reshape(CNT_PAD)
  acc1 = _edge_kernel(y1.reshape(NUM_REL * N, FD), gidx, didx, cidx, rc)

  y2, r2 = _layer_dense(r1, acc1, Wrel2, Wroot2, brg2r)
  acc2 = _edge_kernel(y2.reshape(NUM_REL * N, FD), gidx, didx, cidx, rc)

  g = _final_gather_kernel(r2, acc2, idxp)
  out = _head(g, W3, b3r)
  return out[:IDX]
